# baseline (speedup 1.0000x reference)
import functools
import math

import jax
import jax.numpy as jnp
import numpy as np
from jax.experimental import pallas as pl
from jax.experimental.pallas import tpu as pltpu

HID = 768
INTER = 1024
HEADS = 12
DH = 64
FACTOR = 5


def _sample_consts():
    cpu = jax.local_devices(backend="cpu")[0]
    out = []
    with jax.default_device(cpu):
        key = jax.random.key(42)
        k0, k1 = jax.random.split(key)
        for k, L in ((k0, 2048), (k1, 1024)):
            sample_k = min(FACTOR * int(math.ceil(math.log(L))), L)
            u = min(FACTOR * int(math.ceil(math.log(L))), L)
            idx = np.asarray(jax.random.randint(k, (L, sample_k), 0, L))
            cnt = np.zeros((L, L), np.float32)
            np.add.at(cnt, (np.arange(L)[:, None], idx), 1.0)
            maskadd = np.where(cnt > 0.0, 0.0, -np.inf).astype(np.float32)
            cnt_b = np.asarray(jnp.asarray(cnt, jnp.bfloat16))
            mask_b = np.asarray(jnp.asarray(maskadd, jnp.bfloat16))
            out.append((cnt_b, mask_b, u))
    return out


(_CNT0, _MSK0, _U0), (_CNT1, _MSK1, _U1) = _sample_consts()

_BF = jnp.bfloat16
_F32 = jnp.float32


def _bf(a):
    return a.astype(_BF)


def _qkv_kernel(x_ref, cnt_ref, wq_ref, wk_ref, wv_ref, bq_ref, bk_ref,
                bv_ref, q_ref, kt_ref, v_ref, ks_ref):
    x = _bf(x_ref[...])
    q = jnp.dot(x, _bf(wq_ref[0]), preferred_element_type=_F32) + bq_ref[0]
    k = jnp.dot(x, _bf(wk_ref[0]), preferred_element_type=_F32) + bk_ref[0]
    v = jnp.dot(x, _bf(wv_ref[0]), preferred_element_type=_F32) + bv_ref[0]
    q_ref[0] = _bf(q)
    kb = _bf(k)
    kt_ref[0] = kb.T
    v_ref[0] = _bf(v)
    ks_ref[0] = _bf(jnp.dot(cnt_ref[...], kb, preferred_element_type=_F32))


def _m_kernel(q_ref, kt_ref, ksum_ref, msk_ref, m_ref, *, LK):
    s = jnp.dot(q_ref[0], kt_ref[0], preferred_element_type=_F32)
    mx = jnp.max(s + msk_ref[...].astype(_F32), axis=1)
    sm = jnp.sum(q_ref[0].astype(_F32) * ksum_ref[0].astype(_F32), axis=1)
    m_ref[0, 0, 0] = mx - sm * (1.0 / LK)


def _topk_kernel(m_ref, top_ref, *, U):
    m = m_ref[...]
    L = m.shape[1]
    iota = jax.lax.broadcasted_iota(jnp.int32, m.shape, 1)
    for u in range(U):
        mx = jnp.max(m, axis=1, keepdims=True)
        amax = jnp.min(jnp.where(m == mx, iota, L), axis=1, keepdims=True)
        top_ref[:, u:u + 1] = amax
        m = jnp.where(iota == amax, -jnp.inf, m)


def _tail_kernel(q_ref, kt_ref, v_ref, topc_ref, topr_ref, wo_ref, bo_ref,
                 x_ref, g_ref, b_ref, o_ref, d_ref, rv_ref,
                 *, U, UP, scale, NH):
    h = pl.program_id(0)

    @pl.when(h == 0)
    def _():
        rv_ref[...] = jnp.zeros_like(rv_ref)

    q = q_ref[0]
    kt = kt_ref[0]
    v = v_ref[0]
    L = q.shape[0]
    pt = (jax.lax.broadcasted_iota(jnp.int32, (U, L), 1)
          == topc_ref[0]).astype(_BF)
    qr = jnp.dot(pt, q, preferred_element_type=_F32)
    sc = jnp.dot(_bf(qr), kt, preferred_element_type=_F32) * scale
    sc = sc - jnp.max(sc, axis=1, keepdims=True)
    e = jnp.exp(sc)
    attn = e * (1.0 / jnp.sum(e, axis=1, keepdims=True))
    upd = jnp.dot(_bf(attn), v, preferred_element_type=_F32)
    mv = jnp.mean(v.astype(_F32), axis=0, keepdims=True)
    wo = _bf(wo_ref[0])
    d = jnp.dot(_bf(upd - mv), wo, preferred_element_type=_F32)
    if UP > U:
        d = jnp.concatenate(
            [d, jnp.zeros((UP - U, d.shape[1]), _F32)], axis=0)
    d_ref[pl.ds(pl.multiple_of(h * UP, 8), UP), :] = d
    rv_ref[...] += jnp.dot(_bf(mv), wo, preferred_element_type=_F32)

    @pl.when(h == NH - 1)
    def _():
        ptt = (jax.lax.broadcasted_iota(jnp.int32, (L, NH * UP), 1)
               == topr_ref[0]).astype(_BF)
        o = (x_ref[...] + bo_ref[...] + rv_ref[...]
             + jnp.dot(ptt, _bf(d_ref[...]), preferred_element_type=_F32))
        mu = jnp.mean(o, axis=1, keepdims=True)
        var = jnp.mean((o - mu) ** 2, axis=1, keepdims=True)
        o_ref[...] = (o - mu) * jax.lax.rsqrt(var + 1e-5) * g_ref[...] + b_ref[...]


def _ffn_kernel(x_ref, w1_ref, b1_ref, w2_ref, b2_ref, g2_ref, be2_ref,
                gn_ref, bn_ref, o_ref, *, final):
    x = x_ref[...]
    hdn = jnp.maximum(
        jnp.dot(_bf(x), _bf(w1_ref[...]), preferred_element_type=_F32)
        + b1_ref[...], 0.0)
    y = x + jnp.dot(_bf(hdn), _bf(w2_ref[...]), preferred_element_type=_F32) \
        + b2_ref[...]
    mu = jnp.mean(y, axis=1, keepdims=True)
    var = jnp.mean((y - mu) ** 2, axis=1, keepdims=True)
    y = (y - mu) * jax.lax.rsqrt(var + 1e-5) * g2_ref[...] + be2_ref[...]
    if final:
        mu = jnp.mean(y, axis=1, keepdims=True)
        var = jnp.mean((y - mu) ** 2, axis=1, keepdims=True)
        y = (y - mu) * jax.lax.rsqrt(var + 1e-5) * gn_ref[...] + bn_ref[...]
    o_ref[...] = y


def _distill_kernel(xp_ref, w_ref, cb_ref, bng_ref, bnb_ref, o_ref, *, L):
    w = _bf(w_ref[...])
    xp = _bf(xp_ref[...])
    y = (jnp.dot(xp[0:L, :], w[0], preferred_element_type=_F32)
         + jnp.dot(xp[1:L + 1, :], w[1], preferred_element_type=_F32)
         + jnp.dot(xp[2:L + 2, :], w[2], preferred_element_type=_F32)
         + cb_ref[...])
    mu = jnp.mean(y, axis=0, keepdims=True)
    var = jnp.mean((y - mu) ** 2, axis=0, keepdims=True)
    y = (y - mu) * jax.lax.rsqrt(var + 1e-5) * bng_ref[...] + bnb_ref[...]
    y = jnp.where(y > 0.0, y, jnp.exp(y) - 1.0)
    ninf = jnp.full((1, y.shape[1]), -jnp.inf, jnp.float32)
    ym1 = jnp.concatenate([ninf, y[:L - 1]], axis=0)
    yp1 = jnp.concatenate([y[1:], ninf], axis=0)
    o_ref[...] = jnp.maximum(jnp.maximum(ym1, y), yp1)


def _attn_layer(x2, p, cnt, msk, U):
    L = x2.shape[0]

    def _wT3(w):
        return w.T.reshape(HID, HEADS, DH).transpose(1, 0, 2)

    def _b3(b):
        return b.reshape(HEADS, 1, DH)

    q, kt, v, ksum = pl.pallas_call(
        _qkv_kernel,
        grid=(HEADS,),
        in_specs=[
            pl.BlockSpec((L, HID), lambda h: (0, 0)),
            pl.BlockSpec((L, L), lambda h: (0, 0)),
            pl.BlockSpec((1, HID, DH), lambda h: (h, 0, 0)),
            pl.BlockSpec((1, HID, DH), lambda h: (h, 0, 0)),
            pl.BlockSpec((1, HID, DH), lambda h: (h, 0, 0)),
            pl.BlockSpec((1, 1, DH), lambda h: (h, 0, 0)),
            pl.BlockSpec((1, 1, DH), lambda h: (h, 0, 0)),
            pl.BlockSpec((1, 1, DH), lambda h: (h, 0, 0)),
        ],
        out_specs=[
            pl.BlockSpec((1, L, DH), lambda h: (h, 0, 0)),
            pl.BlockSpec((1, DH, L), lambda h: (h, 0, 0)),
            pl.BlockSpec((1, L, DH), lambda h: (h, 0, 0)),
            pl.BlockSpec((1, L, DH), lambda h: (h, 0, 0)),
        ],
        out_shape=[
            jax.ShapeDtypeStruct((HEADS, L, DH), _BF),
            jax.ShapeDtypeStruct((HEADS, DH, L), _BF),
            jax.ShapeDtypeStruct((HEADS, L, DH), _BF),
            jax.ShapeDtypeStruct((HEADS, L, DH), _BF),
        ],
    )(x2, cnt, _wT3(p["Wq"]), _wT3(p["Wk"]), _wT3(p["Wv"]),
      _b3(p["bq"]), _b3(p["bk"]), _b3(p["bv"]))

    BQ = 512
    nqb = L // BQ
    m = pl.pallas_call(
        functools.partial(_m_kernel, LK=L),
        grid=(nqb, HEADS),
        in_specs=[
            pl.BlockSpec((1, BQ, DH), lambda qb, h: (h, qb, 0)),
            pl.BlockSpec((1, DH, L), lambda qb, h: (h, 0, 0)),
            pl.BlockSpec((1, BQ, DH), lambda qb, h: (h, qb, 0)),
            pl.BlockSpec((BQ, L), lambda qb, h: (qb, 0)),
        ],
        out_specs=pl.BlockSpec((1, 1, 1, BQ), lambda qb, h: (h, qb, 0, 0)),
        out_shape=jax.ShapeDtypeStruct((HEADS, nqb, 1, BQ), _F32),
    )(q, kt, ksum, msk)
    m = m.reshape(HEADS, L)

    top = pl.pallas_call(
        functools.partial(_topk_kernel, U=U),
        out_shape=jax.ShapeDtypeStruct((HEADS, U), jnp.int32),
    )(m)
    UP = (U + 7) // 8 * 8
    topc = top.reshape(HEADS, U, 1)
    topr = jnp.pad(top, ((0, 0), (0, UP - U)),
                   constant_values=-1).reshape(1, 1, HEADS * UP)

    woT3 = p["Wo"].T.reshape(HEADS, DH, HID)
    out1 = pl.pallas_call(
        functools.partial(_tail_kernel, U=U, UP=UP,
                          scale=1.0 / math.sqrt(DH), NH=HEADS),
        grid=(HEADS,),
        in_specs=[
            pl.BlockSpec((1, L, DH), lambda h: (h, 0, 0)),
            pl.BlockSpec((1, DH, L), lambda h: (h, 0, 0)),
            pl.BlockSpec((1, L, DH), lambda h: (h, 0, 0)),
            pl.BlockSpec((1, U, 1), lambda h: (h, 0, 0)),
            pl.BlockSpec((1, 1, HEADS * UP), lambda h: (0, 0, 0)),
            pl.BlockSpec((1, DH, HID), lambda h: (h, 0, 0)),
            pl.BlockSpec((1, HID), lambda h: (0, 0)),
            pl.BlockSpec((L, HID), lambda h: (0, 0)),
            pl.BlockSpec((1, HID), lambda h: (0, 0)),
            pl.BlockSpec((1, HID), lambda h: (0, 0)),
        ],
        out_specs=pl.BlockSpec((L, HID), lambda h: (0, 0)),
        out_shape=jax.ShapeDtypeStruct((L, HID), _F32),
        scratch_shapes=[
            pltpu.VMEM((HEADS * UP, HID), _F32),
            pltpu.VMEM((1, HID), _F32),
        ],
    )(q, kt, v, topc, topr, woT3, p["bo"].reshape(1, HID), x2,
      p["g1"].reshape(1, HID), p["be1"].reshape(1, HID))
    return out1


def _ffn(x2, p, final, gn, bn):
    L = x2.shape[0]
    return pl.pallas_call(
        functools.partial(_ffn_kernel, final=final),
        out_shape=jax.ShapeDtypeStruct((L, HID), _F32),
    )(x2, p["W1"].T, p["b1"].reshape(1, INTER), p["W2"].T,
      p["b2"].reshape(1, HID), p["g2"].reshape(1, HID),
      p["be2"].reshape(1, HID), gn.reshape(1, HID), bn.reshape(1, HID))


def _distill(x2, p):
    L = x2.shape[0]
    xp = jnp.concatenate([x2[-1:], x2, x2[:1]], axis=0)
    wT = jnp.transpose(p["convW"], (2, 1, 0))
    b = pl.pallas_call(
        functools.partial(_distill_kernel, L=L),
        out_shape=jax.ShapeDtypeStruct((L, HID), _F32),
    )(xp, wT, p["convb"].reshape(1, HID), p["bng"].reshape(1, HID),
      p["bnb"].reshape(1, HID))
    return b[::2]


def kernel(x, params):
    x2 = x[0]
    x2 = _attn_layer(x2, params["layer0"], _CNT0, _MSK0, _U0)
    x2 = _ffn(x2, params["layer0"], False, params["gN"], params["bN"])
    x2 = _distill(x2, params["distill"])
    x2 = _attn_layer(x2, params["layer1"], _CNT1, _MSK1, _U1)
    x2 = _ffn(x2, params["layer1"], True, params["gN"], params["bN"])
    return x2[None]


# speedup vs baseline: 1.7250x; 1.7250x over previous
import functools
import math

import jax
import jax.numpy as jnp
import numpy as np
from jax.experimental import pallas as pl
from jax.experimental.pallas import tpu as pltpu

HID = 768
INTER = 1024
HEADS = 12
DH = 64
FACTOR = 5

_BF = jnp.bfloat16
_F32 = jnp.float32


def _sample_consts():
    key = jax.random.key(42)
    k0, k1 = jax.random.split(key)
    out = []
    for k, L in ((k0, 2048), (k1, 1024)):
        sample_k = min(FACTOR * int(math.ceil(math.log(L))), L)
        u = min(FACTOR * int(math.ceil(math.log(L))), L)
        idx = np.asarray(jax.random.randint(k, (L, sample_k), 0, L))
        cnt = np.zeros((L, L), np.float32)
        np.add.at(cnt, (np.arange(L)[:, None], idx), 1.0)
        out.append((jnp.asarray(cnt, _BF), u))
    return out


(_CNT0, _U0), (_CNT1, _U1) = _sample_consts()


def _bf(a):
    return a.astype(_BF)


def _ln(y, g, b):
    mu = jnp.mean(y, axis=1, keepdims=True)
    var = jnp.mean((y - mu) ** 2, axis=1, keepdims=True)
    return (y - mu) * jax.lax.rsqrt(var + 1e-5) * g + b


def _layer_kernel(x_ref, cnt_ref, wq_ref, wk_ref, wv_ref, wot_ref,
                  bq_ref, bk_ref, bv_ref, bo_ref, g1_ref, be1_ref,
                  w1t_ref, b1_ref, w2t_ref, b2_ref, g2_ref, be2_ref,
                  gn_ref, bn_ref, o_ref,
                  qs, kts, vs, ms, tops, topflat, ds, hs,
                  *, L, U, UP, BQ, final):
    nqb = L // BQ
    scale = 1.0 / math.sqrt(DH)

    xb = _bf(x_ref[...])
    qs[...] = _bf(
        jax.lax.dot_general(xb, wq_ref[...], (((1,), (1,)), ((), ())),
                            preferred_element_type=_F32) + bq_ref[...])
    kb = _bf(
        jax.lax.dot_general(xb, wk_ref[...], (((1,), (1,)), ((), ())),
                            preferred_element_type=_F32) + bk_ref[...])
    kts[...] = kb.T
    vs[...] = _bf(
        jax.lax.dot_general(xb, wv_ref[...], (((1,), (1,)), ((), ())),
                            preferred_element_type=_F32) + bv_ref[...])

    for h in range(HEADS):
        kt_h = kts[h * DH:(h + 1) * DH, :]
        for qb in range(nqb):
            qv = qs[qb * BQ:(qb + 1) * BQ, h * DH:(h + 1) * DH]
            s = jnp.dot(qv, kt_h, preferred_element_type=_F32)
            c = cnt_ref[qb * BQ:(qb + 1) * BQ, :].astype(_F32)
            mx = jnp.max(jnp.where(c > 0.0, s, -jnp.inf), axis=1,
                         keepdims=True)
            sm = jnp.sum(s * c, axis=1, keepdims=True)
            ms[qb * BQ:(qb + 1) * BQ, h:h + 1] = mx - sm * (1.0 / L)

    m = ms[...].T
    iota = jax.lax.broadcasted_iota(jnp.int32, (HEADS, L), 1)
    tops[...] = jnp.full((HEADS, UP), -1, jnp.int32)
    for u in range(U):
        mxv = jnp.max(m, axis=1, keepdims=True)
        amax = jnp.min(jnp.where(m == mxv, iota, L), axis=1, keepdims=True)
        tops[:, u:u + 1] = amax
        m = jnp.where(iota == amax, -jnp.inf, m)
    for h in range(HEADS):
        topflat[0:1, h * UP:(h + 1) * UP] = tops[h:h + 1, :]

    rv = bo_ref[...]
    for h in range(HEADS):
        kt_h = kts[h * DH:(h + 1) * DH, :]
        q_h = qs[:, h * DH:(h + 1) * DH]
        v_h = vs[:, h * DH:(h + 1) * DH]
        ptl = (jax.lax.broadcasted_iota(jnp.int32, (L, U), 0)
               == tops[h:h + 1, 0:U]).astype(_BF)
        pt = ptl.T
        qr = jnp.dot(pt, q_h, preferred_element_type=_F32)
        sc = jnp.dot(_bf(qr), kt_h, preferred_element_type=_F32) * scale
        sc = sc - jnp.max(sc, axis=1, keepdims=True)
        e = jnp.exp(sc)
        attn = e * (1.0 / jnp.sum(e, axis=1, keepdims=True))
        upd = jnp.dot(_bf(attn), v_h, preferred_element_type=_F32)
        mv = jnp.mean(v_h.astype(_F32), axis=0, keepdims=True)
        wo_h = wot_ref[h * DH:(h + 1) * DH, :]
        d = jnp.dot(_bf(upd - mv), wo_h, preferred_element_type=_F32)
        if UP > U:
            d = jnp.concatenate(
                [d, jnp.zeros((UP - U, HID), _F32)], axis=0)
        ds[h * UP:(h + 1) * UP, :] = d
        rv = rv + jnp.dot(_bf(mv), wo_h, preferred_element_type=_F32)

    ptt = (jax.lax.broadcasted_iota(jnp.int32, (L, HEADS * UP), 1)
           == topflat[...]).astype(_BF)
    o = (x_ref[...] + rv
         + jnp.dot(ptt, _bf(ds[...]), preferred_element_type=_F32))
    o = _ln(o, g1_ref[...], be1_ref[...])

    hs[...] = _bf(jnp.maximum(
        jnp.dot(_bf(o), w1t_ref[...], preferred_element_type=_F32)
        + b1_ref[...], 0.0))
    y = o + jnp.dot(hs[...], w2t_ref[...], preferred_element_type=_F32) \
        + b2_ref[...]
    y = _ln(y, g2_ref[...], be2_ref[...])
    if final:
        y = _ln(y, gn_ref[...], bn_ref[...])
    o_ref[...] = y


def _distill_kernel(xp_ref, w_ref, cb_ref, bng_ref, bnb_ref, o_ref, *, L):
    xp = _bf(xp_ref[...])
    dn = (((1,), (1,)), ((), ()))
    y = (jax.lax.dot_general(xp[0:L, :], w_ref[0], dn,
                             preferred_element_type=_F32)
         + jax.lax.dot_general(xp[1:L + 1, :], w_ref[1], dn,
                               preferred_element_type=_F32)
         + jax.lax.dot_general(xp[2:L + 2, :], w_ref[2], dn,
                               preferred_element_type=_F32)
         + cb_ref[...])
    mu = jnp.mean(y, axis=0, keepdims=True)
    var = jnp.mean((y - mu) ** 2, axis=0, keepdims=True)
    y = (y - mu) * jax.lax.rsqrt(var + 1e-5) * bng_ref[...] + bnb_ref[...]
    y = jnp.where(y > 0.0, y, jnp.exp(y) - 1.0)
    ninf = jnp.full((1, y.shape[1]), -jnp.inf, jnp.float32)
    ym1 = jnp.concatenate([ninf, y[:L - 1]], axis=0)
    yp1 = jnp.concatenate([y[1:], ninf], axis=0)
    o_ref[...] = jnp.maximum(jnp.maximum(ym1, y), yp1)


def _attn_ffn_layer(x2, p, cnt, U, final, gn, bn):
    L = x2.shape[0]
    UP = (U + 7) // 8 * 8
    BQ = 512
    row = lambda a: a.reshape(1, -1)
    return pl.pallas_call(
        functools.partial(_layer_kernel, L=L, U=U, UP=UP, BQ=BQ,
                          final=final),
        out_shape=jax.ShapeDtypeStruct((L, HID), _F32),
        scratch_shapes=[
            pltpu.VMEM((L, HID), _BF),
            pltpu.VMEM((HID, L), _BF),
            pltpu.VMEM((L, HID), _BF),
            pltpu.VMEM((L, HEADS), _F32),
            pltpu.VMEM((HEADS, UP), jnp.int32),
            pltpu.VMEM((1, HEADS * UP), jnp.int32),
            pltpu.VMEM((HEADS * UP, HID), _F32),
            pltpu.VMEM((L, INTER), _BF),
        ],
    )(x2, cnt, _bf(p["Wq"]), _bf(p["Wk"]), _bf(p["Wv"]), _bf(p["Wo"].T),
      row(p["bq"]), row(p["bk"]), row(p["bv"]), row(p["bo"]),
      row(p["g1"]), row(p["be1"]),
      _bf(p["W1"].T), row(p["b1"]), _bf(p["W2"].T), row(p["b2"]),
      row(p["g2"]), row(p["be2"]), row(gn), row(bn))


def _distill(x2, p):
    L = x2.shape[0]
    xp = jnp.concatenate([x2[-1:], x2, x2[:1]], axis=0)
    wT = _bf(jnp.transpose(p["convW"], (2, 0, 1)))
    b = pl.pallas_call(
        functools.partial(_distill_kernel, L=L),
        out_shape=jax.ShapeDtypeStruct((L, HID), _F32),
    )(xp, wT, p["convb"].reshape(1, HID), p["bng"].reshape(1, HID),
      p["bnb"].reshape(1, HID))
    return b[::2]


def kernel(x, params):
    x2 = x[0]
    x2 = _attn_ffn_layer(x2, params["layer0"], _CNT0, _U0, False,
                         params["gN"], params["bN"])
    x2 = _distill(x2, params["distill"])
    x2 = _attn_ffn_layer(x2, params["layer1"], _CNT1, _U1, True,
                         params["gN"], params["bN"])
    return x2[None]


# speedup vs baseline: 1.8337x; 1.0630x over previous
import functools
import math

import jax
import jax.numpy as jnp
import numpy as np
from jax.experimental import pallas as pl
from jax.experimental.pallas import tpu as pltpu

HID = 768
INTER = 1024
HEADS = 12
DH = 64
FACTOR = 5

_BF = jnp.bfloat16
_F32 = jnp.float32


def _sample_consts():
    cpu = jax.local_devices(backend="cpu")[0]
    with jax.default_device(cpu):
        key = jax.random.key(42)
        k0, k1 = jax.random.split(key)
        out = []
        for k, L in ((k0, 2048), (k1, 1024)):
            sample_k = min(FACTOR * int(math.ceil(math.log(L))), L)
            u = min(FACTOR * int(math.ceil(math.log(L))), L)
            idx = np.asarray(jax.random.randint(k, (L, sample_k), 0, L))
            cnt = np.zeros((L, L), np.float32)
            np.add.at(cnt, (np.arange(L)[:, None], idx), 1.0)
            out.append((np.asarray(jnp.asarray(cnt, _BF)), u))
    return out


(_CNT0, _U0), (_CNT1, _U1) = _sample_consts()


def _bf(a):
    return a.astype(_BF)


def _ln(y, g, b):
    mu = jnp.mean(y, axis=1, keepdims=True)
    var = jnp.mean((y - mu) ** 2, axis=1, keepdims=True)
    return (y - mu) * jax.lax.rsqrt(var + 1e-5) * g + b


def _layer_kernel(x_ref, cnt_ref, wq_ref, wk_ref, wv_ref, wot_ref,
                  bq_ref, bk_ref, bv_ref, bo_ref, g1_ref, be1_ref,
                  w1t_ref, b1_ref, w2t_ref, b2_ref, g2_ref, be2_ref,
                  gn_ref, bn_ref, o_ref,
                  qs, kts, vs, ms, tops, topflat, ds, hs,
                  *, L, U, UP, BQ, final):
    nqb = L // BQ
    scale = 1.0 / math.sqrt(DH)

    xb = _bf(x_ref[...])
    qs[...] = _bf(
        jax.lax.dot_general(xb, wq_ref[...], (((1,), (1,)), ((), ())),
                            preferred_element_type=_F32) + bq_ref[...])
    kb = _bf(
        jax.lax.dot_general(xb, wk_ref[...], (((1,), (1,)), ((), ())),
                            preferred_element_type=_F32) + bk_ref[...])
    kts[...] = kb.T
    vs[...] = _bf(
        jax.lax.dot_general(xb, wv_ref[...], (((1,), (1,)), ((), ())),
                            preferred_element_type=_F32) + bv_ref[...])

    for qb in range(nqb):
        c = cnt_ref[qb * BQ:(qb + 1) * BQ, :]
        negm = (jnp.minimum(c.astype(_F32), 1.0) - 1.0) * 1e30
        cntk = jax.lax.dot_general(c, kts[...], (((1,), (1,)), ((), ())),
                                   preferred_element_type=_F32)
        for h in range(HEADS):
            kt_h = kts[h * DH:(h + 1) * DH, :]
            qv = qs[qb * BQ:(qb + 1) * BQ, h * DH:(h + 1) * DH]
            s = jnp.dot(qv, kt_h, preferred_element_type=_F32)
            mx = jnp.max(s + negm, axis=1, keepdims=True)
            sm = jnp.sum(qv.astype(_F32) * cntk[:, h * DH:(h + 1) * DH],
                         axis=1, keepdims=True)
            ms[qb * BQ:(qb + 1) * BQ, h:h + 1] = mx - sm * (1.0 / L)

    m = ms[...].T
    iota = jax.lax.broadcasted_iota(jnp.int32, (HEADS, L), 1)
    tops[...] = jnp.full((HEADS, UP), -1, jnp.int32)
    for u in range(U):
        mxv = jnp.max(m, axis=1, keepdims=True)
        amax = jnp.min(jnp.where(m == mxv, iota, L), axis=1, keepdims=True)
        tops[:, u:u + 1] = amax
        m = jnp.where(iota == amax, -jnp.inf, m)
    for h in range(HEADS):
        topflat[0:1, h * UP:(h + 1) * UP] = tops[h:h + 1, :]

    rv = bo_ref[...]
    for h in range(HEADS):
        kt_h = kts[h * DH:(h + 1) * DH, :]
        q_h = qs[:, h * DH:(h + 1) * DH]
        v_h = vs[:, h * DH:(h + 1) * DH]
        ptl = (jax.lax.broadcasted_iota(jnp.int32, (L, U), 0)
               == tops[h:h + 1, 0:U]).astype(_BF)
        pt = ptl.T
        qr = jnp.dot(pt, q_h, preferred_element_type=_F32)
        sc = jnp.dot(_bf(qr), kt_h, preferred_element_type=_F32) * scale
        sc = sc - jnp.max(sc, axis=1, keepdims=True)
        e = jnp.exp(sc)
        attn = e * (1.0 / jnp.sum(e, axis=1, keepdims=True))
        upd = jnp.dot(_bf(attn), v_h, preferred_element_type=_F32)
        mv = jnp.mean(v_h.astype(_F32), axis=0, keepdims=True)
        wo_h = wot_ref[h * DH:(h + 1) * DH, :]
        d = jnp.dot(_bf(upd - mv), wo_h, preferred_element_type=_F32)
        if UP > U:
            d = jnp.concatenate(
                [d, jnp.zeros((UP - U, HID), _F32)], axis=0)
        ds[h * UP:(h + 1) * UP, :] = d
        rv = rv + jnp.dot(_bf(mv), wo_h, preferred_element_type=_F32)

    ptt = (jax.lax.broadcasted_iota(jnp.int32, (L, HEADS * UP), 1)
           == topflat[...]).astype(_BF)
    o = (x_ref[...] + rv
         + jnp.dot(ptt, _bf(ds[...]), preferred_element_type=_F32))
    o = _ln(o, g1_ref[...], be1_ref[...])

    hs[...] = _bf(jnp.maximum(
        jnp.dot(_bf(o), w1t_ref[...], preferred_element_type=_F32)
        + b1_ref[...], 0.0))
    y = o + jnp.dot(hs[...], w2t_ref[...], preferred_element_type=_F32) \
        + b2_ref[...]
    y = _ln(y, g2_ref[...], be2_ref[...])
    if final:
        y = _ln(y, gn_ref[...], bn_ref[...])
    o_ref[...] = y


def _distill_kernel(xp_ref, w_ref, cb_ref, bng_ref, bnb_ref, o_ref, *, L):
    xp = _bf(xp_ref[...])
    dn = (((1,), (1,)), ((), ()))
    y = (jax.lax.dot_general(xp[0:L, :], w_ref[0], dn,
                             preferred_element_type=_F32)
         + jax.lax.dot_general(xp[1:L + 1, :], w_ref[1], dn,
                               preferred_element_type=_F32)
         + jax.lax.dot_general(xp[2:L + 2, :], w_ref[2], dn,
                               preferred_element_type=_F32)
         + cb_ref[...])
    mu = jnp.mean(y, axis=0, keepdims=True)
    var = jnp.mean((y - mu) ** 2, axis=0, keepdims=True)
    y = (y - mu) * jax.lax.rsqrt(var + 1e-5) * bng_ref[...] + bnb_ref[...]
    y = jnp.where(y > 0.0, y, jnp.exp(y) - 1.0)
    ninf = jnp.full((1, y.shape[1]), -jnp.inf, jnp.float32)
    ym1 = jnp.concatenate([ninf, y[:L - 1]], axis=0)
    yp1 = jnp.concatenate([y[1:], ninf], axis=0)
    o_ref[...] = jnp.maximum(jnp.maximum(ym1, y), yp1)


def _attn_ffn_layer(x2, p, cnt, U, final, gn, bn):
    L = x2.shape[0]
    UP = (U + 7) // 8 * 8
    BQ = 512
    row = lambda a: a.reshape(1, -1)
    return pl.pallas_call(
        functools.partial(_layer_kernel, L=L, U=U, UP=UP, BQ=BQ,
                          final=final),
        out_shape=jax.ShapeDtypeStruct((L, HID), _F32),
        scratch_shapes=[
            pltpu.VMEM((L, HID), _BF),
            pltpu.VMEM((HID, L), _BF),
            pltpu.VMEM((L, HID), _BF),
            pltpu.VMEM((L, HEADS), _F32),
            pltpu.VMEM((HEADS, UP), jnp.int32),
            pltpu.VMEM((1, HEADS * UP), jnp.int32),
            pltpu.VMEM((HEADS * UP, HID), _F32),
            pltpu.VMEM((L, INTER), _BF),
        ],
    )(x2, cnt, _bf(p["Wq"]), _bf(p["Wk"]), _bf(p["Wv"]), _bf(p["Wo"].T),
      row(p["bq"]), row(p["bk"]), row(p["bv"]), row(p["bo"]),
      row(p["g1"]), row(p["be1"]),
      _bf(p["W1"].T), row(p["b1"]), _bf(p["W2"].T), row(p["b2"]),
      row(p["g2"]), row(p["be2"]), row(gn), row(bn))


def _distill(x2, p):
    L = x2.shape[0]
    xp = jnp.concatenate([x2[-1:], x2, x2[:1]], axis=0)
    wT = _bf(jnp.transpose(p["convW"], (2, 0, 1)))
    b = pl.pallas_call(
        functools.partial(_distill_kernel, L=L),
        out_shape=jax.ShapeDtypeStruct((L, HID), _F32),
    )(xp, wT, p["convb"].reshape(1, HID), p["bng"].reshape(1, HID),
      p["bnb"].reshape(1, HID))
    return b[::2]


def kernel(x, params):
    x2 = x[0]
    x2 = _attn_ffn_layer(x2, params["layer0"], _CNT0, _U0, False,
                         params["gN"], params["bN"])
    x2 = _distill(x2, params["distill"])
    x2 = _attn_ffn_layer(x2, params["layer1"], _CNT1, _U1, True,
                         params["gN"], params["bN"])
    return x2[None]


# speedup vs baseline: 1.9627x; 1.0703x over previous
import functools
import math

import jax
import jax.numpy as jnp
import numpy as np
from jax.experimental import pallas as pl
from jax.experimental.pallas import tpu as pltpu

HID = 768
INTER = 1024
HEADS = 12
DH = 64
FACTOR = 5

_BF = jnp.bfloat16
_F32 = jnp.float32


def _sample_consts():
    cpu = jax.local_devices(backend="cpu")[0]
    with jax.default_device(cpu):
        key = jax.random.key(42)
        k0, k1 = jax.random.split(key)
        out = []
        for k, L in ((k0, 2048), (k1, 1024)):
            sample_k = min(FACTOR * int(math.ceil(math.log(L))), L)
            u = min(FACTOR * int(math.ceil(math.log(L))), L)
            idx = np.asarray(jax.random.randint(k, (L, sample_k), 0, L))
            cnt = np.zeros((L, L), np.float32)
            np.add.at(cnt, (np.arange(L)[:, None], idx), 1.0)
            out.append((np.asarray(jnp.asarray(cnt, _BF)), u))
    return out


(_CNT0, _U0), (_CNT1, _U1) = _sample_consts()


def _bf(a):
    return a.astype(_BF)


def _ln(y, g, b):
    mu = jnp.mean(y, axis=1, keepdims=True)
    var = jnp.mean((y - mu) ** 2, axis=1, keepdims=True)
    return (y - mu) * jax.lax.rsqrt(var + 1e-5) * g + b


def _layer_kernel(x_ref, cnt_ref, hm_ref, wq_ref, wk_ref, wv_ref, wot_ref,
                  bq_ref, bk_ref, bv_ref, bo_ref, g1_ref, be1_ref,
                  w1t_ref, b1_ref, w2t_ref, b2_ref, g2_ref, be2_ref,
                  gn_ref, bn_ref, o_ref,
                  qs, kts, vs, ms, tops, topflat, us, hs,
                  *, L, U, UP, BQ, final):
    nqb = L // BQ
    scale = 1.0 / math.sqrt(DH)

    xb = _bf(x_ref[...])
    qs[...] = _bf(
        jax.lax.dot_general(xb, wq_ref[...], (((1,), (1,)), ((), ())),
                            preferred_element_type=_F32) + bq_ref[...])
    kb = _bf(
        jax.lax.dot_general(xb, wk_ref[...], (((1,), (1,)), ((), ())),
                            preferred_element_type=_F32) + bk_ref[...])
    kts[...] = kb.T
    vs[...] = _bf(
        jax.lax.dot_general(xb, wv_ref[...], (((1,), (1,)), ((), ())),
                            preferred_element_type=_F32) + bv_ref[...])

    for qb in range(nqb):
        c = cnt_ref[qb * BQ:(qb + 1) * BQ, :]
        negm = (jnp.minimum(c.astype(_F32), 1.0) - 1.0) * 1e30
        cntk = jax.lax.dot_general(c, kts[...], (((1,), (1,)), ((), ())),
                                   preferred_element_type=_F32)
        for h in range(HEADS):
            kt_h = kts[h * DH:(h + 1) * DH, :]
            qv = qs[qb * BQ:(qb + 1) * BQ, h * DH:(h + 1) * DH]
            s = jnp.dot(qv, kt_h, preferred_element_type=_F32)
            mx = jnp.max(s + negm, axis=1, keepdims=True)
            sm = jnp.sum(qv.astype(_F32) * cntk[:, h * DH:(h + 1) * DH],
                         axis=1, keepdims=True)
            ms[qb * BQ:(qb + 1) * BQ, h:h + 1] = mx - sm * (1.0 / L)

    m = ms[...].T
    iota = jax.lax.broadcasted_iota(jnp.int32, (HEADS, L), 1)
    tops[...] = jnp.full((HEADS, UP), -1, jnp.int32)
    for u in range(U):
        mxv = jnp.max(m, axis=1, keepdims=True)
        amax = jnp.min(jnp.where(m == mxv, iota, L), axis=1, keepdims=True)
        tops[:, u:u + 1] = amax
        m = jnp.where(iota == amax, -jnp.inf, m)
    for h in range(HEADS):
        topflat[0:1, h * UP:(h + 1) * UP] = tops[h:h + 1, :]

    ptt = (jax.lax.broadcasted_iota(jnp.int32, (L, HEADS * UP), 1)
           == topflat[...]).astype(_BF)
    qr = jax.lax.dot_general(ptt, qs[...], (((0,), (0,)), ((), ())),
                             preferred_element_type=_F32)
    hmf = hm_ref[...].astype(_F32)
    qrp = _bf(qr * hmf)
    R = HEADS * UP
    RB = R // 2
    for rb in range(0, R, RB):
        sc = jax.lax.dot_general(qrp[rb:rb + RB], kts[...],
                                 (((1,), (0,)), ((), ())),
                                 preferred_element_type=_F32) * scale
        sc = sc - jnp.max(sc, axis=1, keepdims=True)
        e = jnp.exp(sc)
        attn = e * (1.0 / jnp.sum(e, axis=1, keepdims=True))
        us[rb:rb + RB, :] = jnp.dot(_bf(attn), vs[...],
                                    preferred_element_type=_F32)
    mv = jnp.mean(vs[...].astype(_F32), axis=0, keepdims=True)
    d_all = jnp.dot(_bf((us[...] - mv) * hmf), wot_ref[...],
                    preferred_element_type=_F32)
    rv = bo_ref[...] + jnp.dot(_bf(mv), wot_ref[...],
                               preferred_element_type=_F32)
    o = (x_ref[...] + rv
         + jnp.dot(ptt, _bf(d_all), preferred_element_type=_F32))
    o = _ln(o, g1_ref[...], be1_ref[...])

    hs[...] = _bf(jnp.maximum(
        jnp.dot(_bf(o), w1t_ref[...], preferred_element_type=_F32)
        + b1_ref[...], 0.0))
    y = o + jnp.dot(hs[...], w2t_ref[...], preferred_element_type=_F32) \
        + b2_ref[...]
    y = _ln(y, g2_ref[...], be2_ref[...])
    if final:
        y = _ln(y, gn_ref[...], bn_ref[...])
    o_ref[...] = y


def _distill_kernel(xp_ref, w_ref, cb_ref, bng_ref, bnb_ref, o_ref, *, L):
    xp = _bf(xp_ref[...])
    dn = (((1,), (1,)), ((), ()))
    y = (jax.lax.dot_general(xp[0:L, :], w_ref[0], dn,
                             preferred_element_type=_F32)
         + jax.lax.dot_general(xp[1:L + 1, :], w_ref[1], dn,
                               preferred_element_type=_F32)
         + jax.lax.dot_general(xp[2:L + 2, :], w_ref[2], dn,
                               preferred_element_type=_F32)
         + cb_ref[...])
    mu = jnp.mean(y, axis=0, keepdims=True)
    var = jnp.mean((y - mu) ** 2, axis=0, keepdims=True)
    y = (y - mu) * jax.lax.rsqrt(var + 1e-5) * bng_ref[...] + bnb_ref[...]
    y = jnp.where(y > 0.0, y, jnp.exp(y) - 1.0)
    ninf = jnp.full((1, y.shape[1]), -jnp.inf, jnp.float32)
    ym1 = jnp.concatenate([ninf, y[:L - 1]], axis=0)
    yp1 = jnp.concatenate([y[1:], ninf], axis=0)
    o_ref[...] = jnp.maximum(jnp.maximum(ym1, y), yp1)


def _head_mask(UP):
    hm = np.zeros((HEADS * UP, HID), np.float32)
    for h in range(HEADS):
        hm[h * UP:(h + 1) * UP, h * DH:(h + 1) * DH] = 1.0
    return hm


def _attn_ffn_layer(x2, p, cnt, U, final, gn, bn):
    L = x2.shape[0]
    UP = (U + 7) // 8 * 8
    BQ = 512
    row = lambda a: a.reshape(1, -1)
    return pl.pallas_call(
        functools.partial(_layer_kernel, L=L, U=U, UP=UP, BQ=BQ,
                          final=final),
        out_shape=jax.ShapeDtypeStruct((L, HID), _F32),
        scratch_shapes=[
            pltpu.VMEM((L, HID), _BF),
            pltpu.VMEM((HID, L), _BF),
            pltpu.VMEM((L, HID), _BF),
            pltpu.VMEM((L, HEADS), _F32),
            pltpu.VMEM((HEADS, UP), jnp.int32),
            pltpu.VMEM((1, HEADS * UP), jnp.int32),
            pltpu.VMEM((HEADS * UP, HID), _F32),
            pltpu.VMEM((L, INTER), _BF),
        ],
    )(x2, cnt, jnp.asarray(_head_mask(UP), _BF),
      _bf(p["Wq"]), _bf(p["Wk"]), _bf(p["Wv"]), _bf(p["Wo"].T),
      row(p["bq"]), row(p["bk"]), row(p["bv"]), row(p["bo"]),
      row(p["g1"]), row(p["be1"]),
      _bf(p["W1"].T), row(p["b1"]), _bf(p["W2"].T), row(p["b2"]),
      row(p["g2"]), row(p["be2"]), row(gn), row(bn))


def _distill(x2, p):
    L = x2.shape[0]
    xp = jnp.concatenate([x2[-1:], x2, x2[:1]], axis=0)
    wT = _bf(jnp.transpose(p["convW"], (2, 0, 1)))
    b = pl.pallas_call(
        functools.partial(_distill_kernel, L=L),
        out_shape=jax.ShapeDtypeStruct((L, HID), _F32),
    )(xp, wT, p["convb"].reshape(1, HID), p["bng"].reshape(1, HID),
      p["bnb"].reshape(1, HID))
    return b[::2]


def kernel(x, params):
    x2 = x[0]
    x2 = _attn_ffn_layer(x2, params["layer0"], _CNT0, _U0, False,
                         params["gN"], params["bN"])
    x2 = _distill(x2, params["distill"])
    x2 = _attn_ffn_layer(x2, params["layer1"], _CNT1, _U1, True,
                         params["gN"], params["bN"])
    return x2[None]
